# baseline (speedup 1.0000x reference)
import functools

import jax
import jax.numpy as jnp
from jax import lax
from jax.experimental import pallas as pl
from jax.experimental.pallas import tpu as pltpu
from jax.experimental.pallas import tpu_sc as plsc

B = 4096
D_IN = 2048
D_SAE = 16384
K = 64
EPS = 1e-5

_INT_MIN = -2147483648
_INT_MAX = 2147483647
_MANT = 0x7FFFFFFF


def _layernorm(x):
    mu = jnp.mean(x, axis=1, keepdims=True)
    xc = x - mu
    var = jnp.sum(xc * xc, axis=1, keepdims=True) / (D_IN - 1)
    return xc / (jnp.sqrt(var) + EPS)


def _encode_kernel(x_ref, bdec_ref, w_ref, benc_ref, out_ref):
    xp = _layernorm(x_ref[...])
    xin = (xp - bdec_ref[...]).astype(jnp.bfloat16)
    out_ref[...] = (
        jnp.dot(xin, w_ref[...], preferred_element_type=jnp.float32)
        + benc_ref[...]
    )


def _encode(x, W_enc, b_enc, b_dec, bm, bn):
    rows = x.shape[0]
    ni, nj = rows // bm, D_SAE // bn
    return pl.pallas_call(
        _encode_kernel,
        grid=(nj, ni),
        in_specs=[
            pl.BlockSpec((bm, D_IN), lambda j, i: (i, 0)),
            pl.BlockSpec((1, D_IN), lambda j, i: (0, 0)),
            pl.BlockSpec((D_IN, bn), lambda j, i: (0, j)),
            pl.BlockSpec((1, bn), lambda j, i: (0, j)),
        ],
        out_specs=pl.BlockSpec((bm, bn), lambda j, i: (i, j)),
        out_shape=jax.ShapeDtypeStruct((rows, D_SAE), jnp.float32),
    )(x, b_dec.reshape(1, D_IN), W_enc, b_enc.reshape(1, D_SAE))


_SC_NW = 32
_NCAND_CHUNKS = 5


def _key_of(bits):
    return jnp.where(bits >= 0, bits, bits ^ jnp.int32(_MANT))


def _bsearch(count_fn, lo0, hi0):
    def body(_, c):
        lo, hi = c
        span = hi - lo
        mid = lo + (span >> 1) + (span & 1)
        ge = count_fn(mid) >= K
        return (jnp.where(ge, mid, lo), jnp.where(ge, hi, mid - 1))

    lo, _ = lax.fori_loop(0, 31, body, (lo0, hi0))
    return lo


def _sc_select_kernel(rpw, pre_hbm, tau_hbm, rowbuf, gkey, cbuf, ckey,
                      taukey):
    info = plsc.get_sparse_core_info()
    nc = info.num_cores
    wid = lax.axis_index("s") * nc + lax.axis_index("c")
    base = wid * rpw
    i16 = lax.iota(jnp.int32, 16)
    zero16 = jnp.zeros((16,), jnp.int32)
    imin16 = jnp.full((16,), _INT_MIN, jnp.int32)
    imax16 = jnp.full((16,), _INT_MAX, jnp.int32)
    k16 = jnp.full((16,), K, jnp.int32)
    ninf16 = jnp.full((16,), -jnp.inf, jnp.float32)

    def popcnt(mask):
        return plsc.all_reduce_population_count(mask)

    def count_gkey(mid):
        def cb(j, acc):
            for u in range(16):
                acc = acc + popcnt(gkey[pl.ds((j * 16 + u) * 16, 16)] >= mid)
            return acc

        return lax.fori_loop(0, 4, cb, zero16)

    def row_body(r, _):
        pltpu.sync_copy(pre_hbm.at[base + r], rowbuf)

        def gbody(j, _):
            acc = ninf16
            for t in range(16):
                acc = jnp.maximum(acc, rowbuf[pl.ds(j * 16 + 1024 * t, 16)])
            gkey[pl.ds(j * 16, 16)] = _key_of(plsc.bitcast(acc, jnp.int32))
            return 0

        lax.fori_loop(0, 64, gbody, 0)

        cpos = count_gkey(zero16)
        pos = cpos >= k16
        t_lo = _bsearch(count_gkey,
                        jnp.where(pos, zero16, imin16),
                        jnp.where(pos, imax16, jnp.full((16,), -1,
                                                        jnp.int32)))

        def compact_body(j, off):
            v = gkey[pl.ds(j * 16, 16)]
            m = v >= t_lo
            plsc.store_compressed(cbuf.at[pl.ds(off, 16)], i16 + j * 16,
                                  mask=m)
            return off + popcnt(m)[0]

        ncand = lax.fori_loop(0, 64, compact_body, jnp.int32(0))
        nchunk = jnp.minimum((ncand + 15) >> 4, _NCAND_CHUNKS)

        def gath_body(c, coff):
            valid = (i16 + c * 16) < ncand
            ids = jnp.where(valid, cbuf[pl.ds(c * 16, 16)], 0)
            for t in range(16):
                g = plsc.load_gather(rowbuf, [ids + 1024 * t])
                kk = _key_of(plsc.bitcast(g, jnp.int32))
                m = jnp.logical_and(valid, kk >= t_lo)
                plsc.store_compressed(ckey.at[pl.ds(coff, 16)], kk, mask=m)
                coff = coff + popcnt(m)[0]
            return coff

        celems = lax.fori_loop(0, nchunk, gath_body, jnp.int32(0))
        ckey[pl.ds(celems, 16)] = imin16
        nck = (celems + 15) >> 4

        def count_ckey(mid):
            def cb(j, acc):
                return acc + popcnt(ckey[pl.ds(j * 16, 16)] >= mid)

            return lax.fori_loop(0, nck, cb, zero16)

        c0 = count_ckey(zero16)
        tneg = t_lo < 0
        neg = jnp.logical_and(tneg, c0 < k16)
        lo0 = jnp.where(tneg, jnp.where(neg, t_lo, zero16), t_lo)
        hi0 = jnp.where(neg, jnp.full((16,), -1, jnp.int32), imax16)
        tk = _bsearch(count_ckey, lo0, hi0)
        chunk = (r >> 4) * 16
        old = taukey[pl.ds(chunk, 16)]
        taukey[pl.ds(chunk, 16)] = jnp.where(i16 == (r & 15), tk, old)
        return 0

    lax.fori_loop(0, rpw, row_body, 0)
    pltpu.sync_copy(taukey, tau_hbm.at[pl.ds(base, rpw)])


def _sc_select(pre):
    rows = pre.shape[0]
    rpw = rows // _SC_NW
    mesh = plsc.VectorSubcoreMesh(core_axis_name="c", subcore_axis_name="s")
    f = functools.partial(pl.kernel, mesh=mesh,
                          compiler_params=pltpu.CompilerParams(
                              needs_layout_passes=False),
                          out_type=jax.ShapeDtypeStruct((rows,), jnp.int32),
                          scratch_types=[
                              pltpu.VMEM((D_SAE,), jnp.float32),
                              pltpu.VMEM((1024,), jnp.int32),
                              pltpu.VMEM((1040,), jnp.int32),
                              pltpu.VMEM((_NCAND_CHUNKS * 256 + 16,),
                                         jnp.int32),
                              pltpu.VMEM((rpw,), jnp.int32),
                          ])(functools.partial(_sc_select_kernel, rpw))
    tk = f(pre)
    tau_bits = jnp.where(tk >= 0, tk, tk ^ jnp.int32(_MANT))
    return jax.lax.bitcast_convert_type(tau_bits, jnp.float32)


def _decode_kernel(pre_ref, tau_ref, w_ref, x_ref, bdec_ref,
                   out_ref, s_ref, col_ref):
    i = pl.program_id(0)
    k = pl.program_id(1)
    nk = pl.num_programs(1)
    pre = pre_ref[...]
    tau = tau_ref[:, 0:1]
    lat = jnp.where(pre >= tau, jnp.maximum(pre, 0.0), 0.0)
    contrib = jnp.dot(lat.astype(jnp.bfloat16), w_ref[...],
                      preferred_element_type=jnp.float32)

    @pl.when(k == 0)
    def _():
        out_ref[...] = contrib + bdec_ref[...]

    @pl.when(k > 0)
    def _():
        out_ref[...] += contrib

    @pl.when(jnp.logical_and(i == 0, k == 0))
    def _():
        s_ref[...] = jnp.zeros_like(s_ref)
        col_ref[...] = jnp.zeros_like(col_ref)

    @pl.when(k == nk - 1)
    def _():
        xp = _layernorm(x_ref[...])
        diff = out_ref[...] - xp
        s0 = jnp.sum(diff * diff)
        s1 = jnp.sum(xp * xp)
        row = jax.lax.broadcasted_iota(jnp.int32, s_ref.shape, 0)
        s_ref[...] += jnp.where(row == 0, s0, s1) * (row < 2)
        col_ref[...] += jnp.broadcast_to(
            jnp.sum(xp, axis=0, keepdims=True), col_ref.shape)


def _decode(pre, tau, W_dec, x, b_dec, bm, bk):
    rows = pre.shape[0]
    ni, nk = rows // bm, D_SAE // bk
    return pl.pallas_call(
        _decode_kernel,
        grid=(ni, nk),
        in_specs=[
            pl.BlockSpec((bm, bk), lambda i, k: (i, k)),
            pl.BlockSpec((bm, 128), lambda i, k: (i, 0)),
            pl.BlockSpec((bk, D_IN), lambda i, k: (k, 0)),
            pl.BlockSpec((bm, D_IN), lambda i, k: (i, 0)),
            pl.BlockSpec((1, D_IN), lambda i, k: (0, 0)),
        ],
        out_specs=[
            pl.BlockSpec((bm, D_IN), lambda i, k: (i, 0)),
            pl.BlockSpec((8, 128), lambda i, k: (0, 0)),
            pl.BlockSpec((8, D_IN), lambda i, k: (0, 0)),
        ],
        out_shape=[
            jax.ShapeDtypeStruct((rows, D_IN), jnp.float32),
            jax.ShapeDtypeStruct((8, 128), jnp.float32),
            jax.ShapeDtypeStruct((8, D_IN), jnp.float32),
        ],
    )(pre, tau, W_dec, x, b_dec.reshape(1, D_IN))


ENC_BM = 256
ENC_BN = 4096
DEC_BK = 512
PIPE = 8
BCH = B // PIPE
DEC_BM = BCH


def kernel(x, W_enc, W_dec, b_enc, b_dec):
    W_enc_bf = W_enc.astype(jnp.bfloat16)
    W_dec_bf = W_dec.astype(jnp.bfloat16)
    recs, ss, cols = [], [], []
    for i in range(PIPE):
        xi = jax.lax.slice_in_dim(x, i * BCH, (i + 1) * BCH, axis=0)
        pre = _encode(xi, W_enc_bf, b_enc, b_dec, bm=ENC_BM, bn=ENC_BN)
        tau = jnp.broadcast_to(_sc_select(pre)[:, None], (BCH, 128))
        rec, s, col = _decode(pre, tau, W_dec_bf, xi, b_dec,
                              bm=DEC_BM, bk=DEC_BK)
        recs.append(rec)
        ss.append(s)
        cols.append(col)
    recons = jnp.concatenate(recs, axis=0)
    s = sum(ss)
    col = sum(cols)
    s0 = s[0, 0]
    s1 = s[1, 0]
    colsum = col[0]
    denom = jnp.float32(B * D_IN)
    mse = s0 / denom
    mse_naive = (s1 - jnp.sum(colsum * colsum) / B) / denom
    mse_loss = mse / mse_naive
    aux_loss = jnp.asarray(0.0, dtype=jnp.float32)
    loss = mse_loss + aux_loss
    return recons, loss, mse_loss, aux_loss


# speedup vs baseline: 1.0361x; 1.0361x over previous
import functools

import jax
import jax.numpy as jnp
from jax import lax
from jax.experimental import pallas as pl
from jax.experimental.pallas import tpu as pltpu
from jax.experimental.pallas import tpu_sc as plsc

B = 4096
D_IN = 2048
D_SAE = 16384
K = 64
EPS = 1e-5

_INT_MIN = -2147483648
_INT_MAX = 2147483647
_MANT = 0x7FFFFFFF


def _layernorm(x):
    mu = jnp.mean(x, axis=1, keepdims=True)
    xc = x - mu
    var = jnp.sum(xc * xc, axis=1, keepdims=True) / (D_IN - 1)
    return xc / (jnp.sqrt(var) + EPS)


def _encode_kernel(x_ref, bdec_ref, w_ref, benc_ref, out_ref):
    xp = _layernorm(x_ref[...])
    xin = (xp - bdec_ref[...]).astype(jnp.bfloat16)
    out_ref[...] = (
        jnp.dot(xin, w_ref[...], preferred_element_type=jnp.float32)
        + benc_ref[...]
    )


def _encode(x, W_enc, b_enc, b_dec, bm, bn):
    rows = x.shape[0]
    ni, nj = rows // bm, D_SAE // bn
    return pl.pallas_call(
        _encode_kernel,
        grid=(nj, ni),
        in_specs=[
            pl.BlockSpec((bm, D_IN), lambda j, i: (i, 0)),
            pl.BlockSpec((1, D_IN), lambda j, i: (0, 0)),
            pl.BlockSpec((D_IN, bn), lambda j, i: (0, j)),
            pl.BlockSpec((1, bn), lambda j, i: (0, j)),
        ],
        out_specs=pl.BlockSpec((bm, bn), lambda j, i: (i, j)),
        out_shape=jax.ShapeDtypeStruct((rows, D_SAE), jnp.float32),
    )(x, b_dec.reshape(1, D_IN), W_enc, b_enc.reshape(1, D_SAE))


_SC_NW = 32
_NCAND_CHUNKS = 5
_G = 64
_NG = D_SAE // _G
_NGC = _NG // 16


def _key_of(bits):
    return jnp.where(bits >= 0, bits, bits ^ jnp.int32(_MANT))


def _bsearch(count_fn, lo0, hi0):
    def body(_, c):
        lo, hi = c
        span = hi - lo
        mid = lo + (span >> 1) + (span & 1)
        ge = count_fn(mid) >= K
        return (jnp.where(ge, mid, lo), jnp.where(ge, hi, mid - 1))

    lo, _ = lax.fori_loop(0, 31, body, (lo0, hi0))
    return lo


def _sc_select_kernel(rpw, pre_hbm, tau_hbm, rowbuf, gkey, cbuf, ckey,
                      taukey):
    info = plsc.get_sparse_core_info()
    nc = info.num_cores
    wid = lax.axis_index("s") * nc + lax.axis_index("c")
    base = wid * rpw
    i16 = lax.iota(jnp.int32, 16)
    zero16 = jnp.zeros((16,), jnp.int32)
    imin16 = jnp.full((16,), _INT_MIN, jnp.int32)
    imax16 = jnp.full((16,), _INT_MAX, jnp.int32)
    k16 = jnp.full((16,), K, jnp.int32)
    ninf16 = jnp.full((16,), -jnp.inf, jnp.float32)

    def popcnt(mask):
        return plsc.all_reduce_population_count(mask)

    def count_gkey(mid):
        acc = zero16
        for u in range(_NGC):
            acc = acc + popcnt(gkey[pl.ds(u * 16, 16)] >= mid)
        return acc

    def row_body(r, _):
        pltpu.sync_copy(pre_hbm.at[base + r], rowbuf)

        def gbody(j, _):
            acc = ninf16
            for t in range(_G):
                acc = jnp.maximum(acc, rowbuf[pl.ds(j * 16 + _NG * t, 16)])
            gkey[pl.ds(j * 16, 16)] = _key_of(plsc.bitcast(acc, jnp.int32))
            return 0

        lax.fori_loop(0, _NGC, gbody, 0)

        cpos = count_gkey(zero16)
        pos = cpos >= k16
        t_lo = _bsearch(count_gkey,
                        jnp.where(pos, zero16, imin16),
                        jnp.where(pos, imax16, jnp.full((16,), -1,
                                                        jnp.int32)))

        def compact_body(j, off):
            v = gkey[pl.ds(j * 16, 16)]
            m = v >= t_lo
            plsc.store_compressed(cbuf.at[pl.ds(off, 16)], i16 + j * 16,
                                  mask=m)
            return off + popcnt(m)[0]

        ncand = lax.fori_loop(0, _NGC, compact_body, jnp.int32(0))
        nchunk = jnp.minimum((ncand + 15) >> 4, _NCAND_CHUNKS)

        def gath_body(c, coff):
            valid = (i16 + c * 16) < ncand
            ids = jnp.where(valid, cbuf[pl.ds(c * 16, 16)], 0)
            for t in range(_G):
                g = plsc.load_gather(rowbuf, [ids + _NG * t])
                kk = _key_of(plsc.bitcast(g, jnp.int32))
                m = jnp.logical_and(valid, kk >= t_lo)
                plsc.store_compressed(ckey.at[pl.ds(coff, 16)], kk, mask=m)
                coff = coff + popcnt(m)[0]
            return coff

        celems = lax.fori_loop(0, nchunk, gath_body, jnp.int32(0))
        ckey[pl.ds(celems, 16)] = imin16
        nck = (celems + 15) >> 4

        def count_ckey(mid):
            def cb(j, acc):
                return acc + popcnt(ckey[pl.ds(j * 16, 16)] >= mid)

            return lax.fori_loop(0, nck, cb, zero16)

        c0 = count_ckey(zero16)
        tneg = t_lo < 0
        neg = jnp.logical_and(tneg, c0 < k16)
        lo0 = jnp.where(tneg, jnp.where(neg, t_lo, zero16), t_lo)
        hi0 = jnp.where(neg, jnp.full((16,), -1, jnp.int32), imax16)
        tk = _bsearch(count_ckey, lo0, hi0)
        chunk = (r >> 4) * 16
        old = taukey[pl.ds(chunk, 16)]
        taukey[pl.ds(chunk, 16)] = jnp.where(i16 == (r & 15), tk, old)
        return 0

    lax.fori_loop(0, rpw, row_body, 0)
    pltpu.sync_copy(taukey, tau_hbm.at[pl.ds(base, rpw)])


def _sc_select(pre):
    rows = pre.shape[0]
    rpw = rows // _SC_NW
    mesh = plsc.VectorSubcoreMesh(core_axis_name="c", subcore_axis_name="s")
    f = functools.partial(pl.kernel, mesh=mesh,
                          compiler_params=pltpu.CompilerParams(
                              needs_layout_passes=False),
                          out_type=jax.ShapeDtypeStruct((rows,), jnp.int32),
                          scratch_types=[
                              pltpu.VMEM((D_SAE,), jnp.float32),
                              pltpu.VMEM((_NG,), jnp.int32),
                              pltpu.VMEM((_NG + 16,), jnp.int32),
                              pltpu.VMEM((_NCAND_CHUNKS * 16 * _G + 16,),
                                         jnp.int32),
                              pltpu.VMEM((rpw,), jnp.int32),
                          ])(functools.partial(_sc_select_kernel, rpw))
    tk = f(pre)
    tau_bits = jnp.where(tk >= 0, tk, tk ^ jnp.int32(_MANT))
    return jax.lax.bitcast_convert_type(tau_bits, jnp.float32)


def _decode_kernel(pre_ref, tau_ref, w_ref, x_ref, bdec_ref,
                   out_ref, s_ref, col_ref):
    i = pl.program_id(0)
    k = pl.program_id(1)
    nk = pl.num_programs(1)
    pre = pre_ref[...]
    tau = tau_ref[:, 0:1]
    lat = jnp.where(pre >= tau, jnp.maximum(pre, 0.0), 0.0)
    contrib = jnp.dot(lat.astype(jnp.bfloat16), w_ref[...],
                      preferred_element_type=jnp.float32)

    @pl.when(k == 0)
    def _():
        out_ref[...] = contrib + bdec_ref[...]

    @pl.when(k > 0)
    def _():
        out_ref[...] += contrib

    @pl.when(jnp.logical_and(i == 0, k == 0))
    def _():
        s_ref[...] = jnp.zeros_like(s_ref)
        col_ref[...] = jnp.zeros_like(col_ref)

    @pl.when(k == nk - 1)
    def _():
        xp = _layernorm(x_ref[...])
        diff = out_ref[...] - xp
        s0 = jnp.sum(diff * diff)
        s1 = jnp.sum(xp * xp)
        row = jax.lax.broadcasted_iota(jnp.int32, s_ref.shape, 0)
        s_ref[...] += jnp.where(row == 0, s0, s1) * (row < 2)
        col_ref[...] += jnp.broadcast_to(
            jnp.sum(xp, axis=0, keepdims=True), col_ref.shape)


def _decode(pre, tau, W_dec, x, b_dec, bm, bk):
    rows = pre.shape[0]
    ni, nk = rows // bm, D_SAE // bk
    return pl.pallas_call(
        _decode_kernel,
        grid=(ni, nk),
        in_specs=[
            pl.BlockSpec((bm, bk), lambda i, k: (i, k)),
            pl.BlockSpec((bm, 128), lambda i, k: (i, 0)),
            pl.BlockSpec((bk, D_IN), lambda i, k: (k, 0)),
            pl.BlockSpec((bm, D_IN), lambda i, k: (i, 0)),
            pl.BlockSpec((1, D_IN), lambda i, k: (0, 0)),
        ],
        out_specs=[
            pl.BlockSpec((bm, D_IN), lambda i, k: (i, 0)),
            pl.BlockSpec((8, 128), lambda i, k: (0, 0)),
            pl.BlockSpec((8, D_IN), lambda i, k: (0, 0)),
        ],
        out_shape=[
            jax.ShapeDtypeStruct((rows, D_IN), jnp.float32),
            jax.ShapeDtypeStruct((8, 128), jnp.float32),
            jax.ShapeDtypeStruct((8, D_IN), jnp.float32),
        ],
    )(pre, tau, W_dec, x, b_dec.reshape(1, D_IN))


ENC_BM = 256
ENC_BN = 4096
DEC_BK = 512
PIPE = 4
BCH = B // PIPE
DEC_BM = BCH


def kernel(x, W_enc, W_dec, b_enc, b_dec):
    W_enc_bf = W_enc.astype(jnp.bfloat16)
    W_dec_bf = W_dec.astype(jnp.bfloat16)
    recs, ss, cols = [], [], []
    for i in range(PIPE):
        xi = jax.lax.slice_in_dim(x, i * BCH, (i + 1) * BCH, axis=0)
        pre = _encode(xi, W_enc_bf, b_enc, b_dec, bm=ENC_BM, bn=ENC_BN)
        tau = jnp.broadcast_to(_sc_select(pre)[:, None], (BCH, 128))
        rec, s, col = _decode(pre, tau, W_dec_bf, xi, b_dec,
                              bm=DEC_BM, bk=DEC_BK)
        recs.append(rec)
        ss.append(s)
        cols.append(col)
    recons = jnp.concatenate(recs, axis=0)
    s = sum(ss)
    col = sum(cols)
    s0 = s[0, 0]
    s1 = s[1, 0]
    colsum = col[0]
    denom = jnp.float32(B * D_IN)
    mse = s0 / denom
    mse_naive = (s1 - jnp.sum(colsum * colsum) / B) / denom
    mse_loss = mse / mse_naive
    aux_loss = jnp.asarray(0.0, dtype=jnp.float32)
    loss = mse_loss + aux_loss
    return recons, loss, mse_loss, aux_loss


# speedup vs baseline: 1.1202x; 1.0812x over previous
import functools

import jax
import jax.numpy as jnp
from jax import lax
from jax.experimental import pallas as pl
from jax.experimental.pallas import tpu as pltpu
from jax.experimental.pallas import tpu_sc as plsc

B = 4096
D_IN = 2048
D_SAE = 16384
K = 64
EPS = 1e-5

_INT_MIN = -2147483648
_INT_MAX = 2147483647
_MANT = 0x7FFFFFFF


def _layernorm(x):
    mu = jnp.mean(x, axis=1, keepdims=True)
    xc = x - mu
    var = jnp.sum(xc * xc, axis=1, keepdims=True) / (D_IN - 1)
    return xc / (jnp.sqrt(var) + EPS)


def _encode_kernel(x_ref, bdec_ref, w_ref, benc_ref, out_ref):
    xp = _layernorm(x_ref[...])
    xin = (xp - bdec_ref[...]).astype(jnp.bfloat16)
    out_ref[...] = (
        jnp.dot(xin, w_ref[...], preferred_element_type=jnp.float32)
        + benc_ref[...]
    )


def _encode(x, W_enc, b_enc, b_dec, bm, bn):
    rows = x.shape[0]
    ni, nj = rows // bm, D_SAE // bn
    return pl.pallas_call(
        _encode_kernel,
        grid=(nj, ni),
        in_specs=[
            pl.BlockSpec((bm, D_IN), lambda j, i: (i, 0)),
            pl.BlockSpec((1, D_IN), lambda j, i: (0, 0)),
            pl.BlockSpec((D_IN, bn), lambda j, i: (0, j)),
            pl.BlockSpec((1, bn), lambda j, i: (0, j)),
        ],
        out_specs=pl.BlockSpec((bm, bn), lambda j, i: (i, j)),
        out_shape=jax.ShapeDtypeStruct((rows, D_SAE), jnp.float32),
    )(x, b_dec.reshape(1, D_IN), W_enc, b_enc.reshape(1, D_SAE))


_SC_NW = 32
_NCAND_CHUNKS = 5
_G = 32
_NG = D_SAE // _G
_NGC = _NG // 16


def _key_of(bits):
    return jnp.where(bits >= 0, bits, bits ^ jnp.int32(_MANT))


def _bsearch(count_fn, lo0, hi0):
    def body(_, c):
        lo, hi = c
        span = hi - lo
        mid = lo + (span >> 1) + (span & 1)
        ge = count_fn(mid) >= K
        return (jnp.where(ge, mid, lo), jnp.where(ge, hi, mid - 1))

    lo, _ = lax.fori_loop(0, 31, body, (lo0, hi0))
    return lo


def _sc_select_kernel(rpw, pre_hbm, tau_hbm, rowbuf, gkey, cbuf, ckey,
                      taukey):
    info = plsc.get_sparse_core_info()
    nc = info.num_cores
    wid = lax.axis_index("s") * nc + lax.axis_index("c")
    base = wid * rpw
    i16 = lax.iota(jnp.int32, 16)
    zero16 = jnp.zeros((16,), jnp.int32)
    imin16 = jnp.full((16,), _INT_MIN, jnp.int32)
    imax16 = jnp.full((16,), _INT_MAX, jnp.int32)
    k16 = jnp.full((16,), K, jnp.int32)
    ninf16 = jnp.full((16,), -jnp.inf, jnp.float32)

    def popcnt(mask):
        return plsc.all_reduce_population_count(mask)

    def count_gkey(mid):
        acc = zero16
        for u in range(_NGC):
            acc = acc + popcnt(gkey[pl.ds(u * 16, 16)] >= mid)
        return acc

    def row_body(r, _):
        pltpu.sync_copy(pre_hbm.at[base + r], rowbuf)

        def gbody(j, _):
            acc = ninf16
            for t in range(_G):
                acc = jnp.maximum(acc, rowbuf[pl.ds(j * 16 + _NG * t, 16)])
            gkey[pl.ds(j * 16, 16)] = _key_of(plsc.bitcast(acc, jnp.int32))
            return 0

        lax.fori_loop(0, _NGC, gbody, 0)

        cpos = count_gkey(zero16)
        pos = cpos >= k16
        t_lo = _bsearch(count_gkey,
                        jnp.where(pos, zero16, imin16),
                        jnp.where(pos, imax16, jnp.full((16,), -1,
                                                        jnp.int32)))

        def compact_body(j, off):
            v = gkey[pl.ds(j * 16, 16)]
            m = v >= t_lo
            plsc.store_compressed(cbuf.at[pl.ds(off, 16)], i16 + j * 16,
                                  mask=m)
            return off + popcnt(m)[0]

        ncand = lax.fori_loop(0, _NGC, compact_body, jnp.int32(0))
        nchunk = jnp.minimum((ncand + 15) >> 4, _NCAND_CHUNKS)

        def gath_body(c, coff):
            valid = (i16 + c * 16) < ncand
            ids = jnp.where(valid, cbuf[pl.ds(c * 16, 16)], 0)
            for t in range(_G):
                g = plsc.load_gather(rowbuf, [ids + _NG * t])
                kk = _key_of(plsc.bitcast(g, jnp.int32))
                m = jnp.logical_and(valid, kk >= t_lo)
                plsc.store_compressed(ckey.at[pl.ds(coff, 16)], kk, mask=m)
                coff = coff + popcnt(m)[0]
            return coff

        celems = lax.fori_loop(0, nchunk, gath_body, jnp.int32(0))
        ckey[pl.ds(celems, 16)] = imin16
        nck = (celems + 15) >> 4

        def count_ckey(mid):
            def cb(j, acc):
                return acc + popcnt(ckey[pl.ds(j * 16, 16)] >= mid)

            return lax.fori_loop(0, nck, cb, zero16)

        c0 = count_ckey(zero16)
        tneg = t_lo < 0
        neg = jnp.logical_and(tneg, c0 < k16)
        lo0 = jnp.where(tneg, jnp.where(neg, t_lo, zero16), t_lo)
        hi0 = jnp.where(neg, jnp.full((16,), -1, jnp.int32), imax16)
        tk = _bsearch(count_ckey, lo0, hi0)
        chunk = (r >> 4) * 16
        old = taukey[pl.ds(chunk, 16)]
        taukey[pl.ds(chunk, 16)] = jnp.where(i16 == (r & 15), tk, old)
        return 0

    lax.fori_loop(0, rpw, row_body, 0)
    pltpu.sync_copy(taukey, tau_hbm.at[pl.ds(base, rpw)])


def _sc_select(pre):
    rows = pre.shape[0]
    rpw = rows // _SC_NW
    mesh = plsc.VectorSubcoreMesh(core_axis_name="c", subcore_axis_name="s")
    f = functools.partial(pl.kernel, mesh=mesh,
                          compiler_params=pltpu.CompilerParams(
                              needs_layout_passes=False),
                          out_type=jax.ShapeDtypeStruct((rows,), jnp.int32),
                          scratch_types=[
                              pltpu.VMEM((D_SAE,), jnp.float32),
                              pltpu.VMEM((_NG,), jnp.int32),
                              pltpu.VMEM((_NG + 16,), jnp.int32),
                              pltpu.VMEM((_NCAND_CHUNKS * 16 * _G + 16,),
                                         jnp.int32),
                              pltpu.VMEM((rpw,), jnp.int32),
                          ])(functools.partial(_sc_select_kernel, rpw))
    tk = f(pre)
    tau_bits = jnp.where(tk >= 0, tk, tk ^ jnp.int32(_MANT))
    return jax.lax.bitcast_convert_type(tau_bits, jnp.float32)


def _decode_kernel(pre_ref, tau_ref, w_ref, x_ref, bdec_ref,
                   out_ref, s_ref, col_ref):
    i = pl.program_id(0)
    k = pl.program_id(1)
    nk = pl.num_programs(1)
    pre = pre_ref[...]
    tau = tau_ref[:, 0:1]
    lat = jnp.where(pre >= tau, jnp.maximum(pre, 0.0), 0.0)
    contrib = jnp.dot(lat.astype(jnp.bfloat16), w_ref[...],
                      preferred_element_type=jnp.float32)

    @pl.when(k == 0)
    def _():
        out_ref[...] = contrib + bdec_ref[...]

    @pl.when(k > 0)
    def _():
        out_ref[...] += contrib

    @pl.when(jnp.logical_and(i == 0, k == 0))
    def _():
        s_ref[...] = jnp.zeros_like(s_ref)
        col_ref[...] = jnp.zeros_like(col_ref)

    @pl.when(k == nk - 1)
    def _():
        xp = _layernorm(x_ref[...])
        diff = out_ref[...] - xp
        s0 = jnp.sum(diff * diff)
        s1 = jnp.sum(xp * xp)
        row = jax.lax.broadcasted_iota(jnp.int32, s_ref.shape, 0)
        s_ref[...] += jnp.where(row == 0, s0, s1) * (row < 2)
        col_ref[...] += jnp.broadcast_to(
            jnp.sum(xp, axis=0, keepdims=True), col_ref.shape)


def _decode(pre, tau, W_dec, x, b_dec, bm, bk):
    rows = pre.shape[0]
    ni, nk = rows // bm, D_SAE // bk
    return pl.pallas_call(
        _decode_kernel,
        grid=(ni, nk),
        in_specs=[
            pl.BlockSpec((bm, bk), lambda i, k: (i, k)),
            pl.BlockSpec((bm, 128), lambda i, k: (i, 0)),
            pl.BlockSpec((bk, D_IN), lambda i, k: (k, 0)),
            pl.BlockSpec((bm, D_IN), lambda i, k: (i, 0)),
            pl.BlockSpec((1, D_IN), lambda i, k: (0, 0)),
        ],
        out_specs=[
            pl.BlockSpec((bm, D_IN), lambda i, k: (i, 0)),
            pl.BlockSpec((8, 128), lambda i, k: (0, 0)),
            pl.BlockSpec((8, D_IN), lambda i, k: (0, 0)),
        ],
        out_shape=[
            jax.ShapeDtypeStruct((rows, D_IN), jnp.float32),
            jax.ShapeDtypeStruct((8, 128), jnp.float32),
            jax.ShapeDtypeStruct((8, D_IN), jnp.float32),
        ],
    )(pre, tau, W_dec, x, b_dec.reshape(1, D_IN))


ENC_BM = 256
ENC_BN = 4096
DEC_BK = 512
PIPE = 4
BCH = B // PIPE
DEC_BM = BCH


def kernel(x, W_enc, W_dec, b_enc, b_dec):
    W_enc_bf = W_enc.astype(jnp.bfloat16)
    W_dec_bf = W_dec.astype(jnp.bfloat16)
    recs, ss, cols = [], [], []
    for i in range(PIPE):
        xi = jax.lax.slice_in_dim(x, i * BCH, (i + 1) * BCH, axis=0)
        pre = _encode(xi, W_enc_bf, b_enc, b_dec, bm=ENC_BM, bn=ENC_BN)
        tau = jnp.broadcast_to(_sc_select(pre)[:, None], (BCH, 128))
        rec, s, col = _decode(pre, tau, W_dec_bf, xi, b_dec,
                              bm=DEC_BM, bk=DEC_BK)
        recs.append(rec)
        ss.append(s)
        cols.append(col)
    recons = jnp.concatenate(recs, axis=0)
    s = sum(ss)
    col = sum(cols)
    s0 = s[0, 0]
    s1 = s[1, 0]
    colsum = col[0]
    denom = jnp.float32(B * D_IN)
    mse = s0 / denom
    mse_naive = (s1 - jnp.sum(colsum * colsum) / B) / denom
    mse_loss = mse / mse_naive
    aux_loss = jnp.asarray(0.0, dtype=jnp.float32)
    loss = mse_loss + aux_loss
    return recons, loss, mse_loss, aux_loss


# speedup vs baseline: 1.1442x; 1.0214x over previous
import functools

import jax
import jax.numpy as jnp
from jax import lax
from jax.experimental import pallas as pl
from jax.experimental.pallas import tpu as pltpu
from jax.experimental.pallas import tpu_sc as plsc

B = 4096
D_IN = 2048
D_SAE = 16384
K = 64
EPS = 1e-5

_INT_MIN = -2147483648
_INT_MAX = 2147483647
_MANT = 0x7FFFFFFF


def _layernorm(x):
    mu = jnp.mean(x, axis=1, keepdims=True)
    xc = x - mu
    var = jnp.sum(xc * xc, axis=1, keepdims=True) / (D_IN - 1)
    return xc / (jnp.sqrt(var) + EPS)


def _encode_kernel(x_ref, bdec_ref, w_ref, benc_ref, out_ref):
    xp = _layernorm(x_ref[...])
    xin = (xp - bdec_ref[...]).astype(jnp.bfloat16)
    out_ref[...] = (
        jnp.dot(xin, w_ref[...], preferred_element_type=jnp.float32)
        + benc_ref[...]
    )


def _encode(x, W_enc, b_enc, b_dec, bm, bn):
    rows = x.shape[0]
    ni, nj = rows // bm, D_SAE // bn
    return pl.pallas_call(
        _encode_kernel,
        grid=(nj, ni),
        in_specs=[
            pl.BlockSpec((bm, D_IN), lambda j, i: (i, 0)),
            pl.BlockSpec((1, D_IN), lambda j, i: (0, 0)),
            pl.BlockSpec((D_IN, bn), lambda j, i: (0, j)),
            pl.BlockSpec((1, bn), lambda j, i: (0, j)),
        ],
        out_specs=pl.BlockSpec((bm, bn), lambda j, i: (i, j)),
        out_shape=jax.ShapeDtypeStruct((rows, D_SAE), jnp.float32),
    )(x, b_dec.reshape(1, D_IN), W_enc, b_enc.reshape(1, D_SAE))


_SC_NW = 32
_NCAND_CHUNKS = 5
_G = 32
_NG = D_SAE // _G
_NGC = _NG // 16


def _key_of(bits):
    return jnp.where(bits >= 0, bits, bits ^ jnp.int32(_MANT))


def _bsearch(count_fn, lo0, hi0):
    def body(_, c):
        lo, hi = c
        span = hi - lo
        mid = lo + (span >> 1) + (span & 1)
        ge = count_fn(mid) >= K
        return (jnp.where(ge, mid, lo), jnp.where(ge, hi, mid - 1))

    lo, _ = lax.fori_loop(0, 31, body, (lo0, hi0))
    return lo


def _sc_select_kernel(rpw, pre_hbm, tau_hbm, rowbuf, gkey, cbuf, ckey,
                      taukey):
    info = plsc.get_sparse_core_info()
    nc = info.num_cores
    wid = lax.axis_index("s") * nc + lax.axis_index("c")
    base = wid * rpw
    i16 = lax.iota(jnp.int32, 16)
    zero16 = jnp.zeros((16,), jnp.int32)
    imin16 = jnp.full((16,), _INT_MIN, jnp.int32)
    imax16 = jnp.full((16,), _INT_MAX, jnp.int32)
    k16 = jnp.full((16,), K, jnp.int32)
    ninf16 = jnp.full((16,), -jnp.inf, jnp.float32)

    def popcnt(mask):
        return plsc.all_reduce_population_count(mask)

    def count_gkey(mid):
        acc = zero16
        for u in range(_NGC):
            acc = acc + popcnt(gkey[pl.ds(u * 16, 16)] >= mid)
        return acc

    def row_body(r, _):
        pltpu.sync_copy(pre_hbm.at[base + r], rowbuf)

        def gbody(j, _):
            acc = ninf16
            for t in range(_G):
                acc = jnp.maximum(acc, rowbuf[pl.ds(j * 16 + _NG * t, 16)])
            gkey[pl.ds(j * 16, 16)] = _key_of(plsc.bitcast(acc, jnp.int32))
            return 0

        lax.fori_loop(0, _NGC, gbody, 0)

        cpos = count_gkey(zero16)
        pos = cpos >= k16
        t_lo = _bsearch(count_gkey,
                        jnp.where(pos, zero16, imin16),
                        jnp.where(pos, imax16, jnp.full((16,), -1,
                                                        jnp.int32)))

        def compact_body(j, off):
            v = gkey[pl.ds(j * 16, 16)]
            m = v >= t_lo
            plsc.store_compressed(cbuf.at[pl.ds(off, 16)], i16 + j * 16,
                                  mask=m)
            return off + popcnt(m)[0]

        ncand = lax.fori_loop(0, _NGC, compact_body, jnp.int32(0))
        nchunk = jnp.minimum((ncand + 15) >> 4, _NCAND_CHUNKS)

        def gath_body(c, coff):
            valid = (i16 + c * 16) < ncand
            ids = jnp.where(valid, cbuf[pl.ds(c * 16, 16)], 0)
            for t in range(_G):
                g = plsc.load_gather(rowbuf, [ids + _NG * t])
                kk = _key_of(plsc.bitcast(g, jnp.int32))
                m = jnp.logical_and(valid, kk >= t_lo)
                plsc.store_compressed(ckey.at[pl.ds(coff, 16)], kk, mask=m)
                coff = coff + popcnt(m)[0]
            return coff

        celems = lax.fori_loop(0, nchunk, gath_body, jnp.int32(0))
        ckey[pl.ds(celems, 16)] = imin16
        nck = (celems + 15) >> 4

        def count_ckey(mid):
            def cb(j, acc):
                return acc + popcnt(ckey[pl.ds(j * 16, 16)] >= mid)

            return lax.fori_loop(0, nck, cb, zero16)

        c0 = count_ckey(zero16)
        tneg = t_lo < 0
        neg = jnp.logical_and(tneg, c0 < k16)
        lo0 = jnp.where(tneg, jnp.where(neg, t_lo, zero16), t_lo)
        hi0 = jnp.where(neg, jnp.full((16,), -1, jnp.int32), imax16)
        tk = _bsearch(count_ckey, lo0, hi0)
        chunk = (r >> 4) * 16
        old = taukey[pl.ds(chunk, 16)]
        taukey[pl.ds(chunk, 16)] = jnp.where(i16 == (r & 15), tk, old)
        return 0

    lax.fori_loop(0, rpw, row_body, 0)
    pltpu.sync_copy(taukey, tau_hbm.at[pl.ds(base, rpw)])


def _sc_select(pre):
    rows = pre.shape[0]
    rpw = rows // _SC_NW
    mesh = plsc.VectorSubcoreMesh(core_axis_name="c", subcore_axis_name="s")
    f = functools.partial(pl.kernel, mesh=mesh,
                          compiler_params=pltpu.CompilerParams(
                              needs_layout_passes=False),
                          out_type=jax.ShapeDtypeStruct((rows,), jnp.int32),
                          scratch_types=[
                              pltpu.VMEM((D_SAE,), jnp.float32),
                              pltpu.VMEM((_NG,), jnp.int32),
                              pltpu.VMEM((_NG + 16,), jnp.int32),
                              pltpu.VMEM((_NCAND_CHUNKS * 16 * _G + 16,),
                                         jnp.int32),
                              pltpu.VMEM((rpw,), jnp.int32),
                          ])(functools.partial(_sc_select_kernel, rpw))
    tk = f(pre)
    tau_bits = jnp.where(tk >= 0, tk, tk ^ jnp.int32(_MANT))
    return jax.lax.bitcast_convert_type(tau_bits, jnp.float32)


def _decode_kernel(pre_ref, tau_ref, w_ref, x_ref, bdec_ref,
                   out_ref, s_ref, col_ref):
    i = pl.program_id(0)
    k = pl.program_id(1)
    nk = pl.num_programs(1)
    pre = pre_ref[...]
    tau = tau_ref[:, 0:1]
    lat = jnp.where(pre >= tau, jnp.maximum(pre, 0.0), 0.0)
    contrib = jnp.dot(lat.astype(jnp.bfloat16), w_ref[...],
                      preferred_element_type=jnp.float32)

    @pl.when(k == 0)
    def _():
        out_ref[...] = contrib + bdec_ref[...]

    @pl.when(k > 0)
    def _():
        out_ref[...] += contrib

    @pl.when(jnp.logical_and(i == 0, k == 0))
    def _():
        s_ref[...] = jnp.zeros_like(s_ref)
        col_ref[...] = jnp.zeros_like(col_ref)

    @pl.when(k == nk - 1)
    def _():
        xp = _layernorm(x_ref[...])
        diff = out_ref[...] - xp
        s0 = jnp.sum(diff * diff)
        s1 = jnp.sum(xp * xp)
        row = jax.lax.broadcasted_iota(jnp.int32, s_ref.shape, 0)
        s_ref[...] += jnp.where(row == 0, s0, s1) * (row < 2)
        col_ref[...] += jnp.broadcast_to(
            jnp.sum(xp, axis=0, keepdims=True), col_ref.shape)


def _decode(pre, tau, W_dec, x, b_dec, bm, bk):
    rows = pre.shape[0]
    ni, nk = rows // bm, D_SAE // bk
    return pl.pallas_call(
        _decode_kernel,
        grid=(ni, nk),
        in_specs=[
            pl.BlockSpec((bm, bk), lambda i, k: (i, k)),
            pl.BlockSpec((bm, 128), lambda i, k: (i, 0)),
            pl.BlockSpec((bk, D_IN), lambda i, k: (k, 0)),
            pl.BlockSpec((bm, D_IN), lambda i, k: (i, 0)),
            pl.BlockSpec((1, D_IN), lambda i, k: (0, 0)),
        ],
        out_specs=[
            pl.BlockSpec((bm, D_IN), lambda i, k: (i, 0)),
            pl.BlockSpec((8, 128), lambda i, k: (0, 0)),
            pl.BlockSpec((8, D_IN), lambda i, k: (0, 0)),
        ],
        out_shape=[
            jax.ShapeDtypeStruct((rows, D_IN), jnp.float32),
            jax.ShapeDtypeStruct((8, 128), jnp.float32),
            jax.ShapeDtypeStruct((8, D_IN), jnp.float32),
        ],
    )(pre, tau, W_dec, x, b_dec.reshape(1, D_IN))


ENC_BM = 256
ENC_BN = 4096
DEC_BK = 1024
PIPE = 4
BCH = B // PIPE
DEC_BM = BCH


def kernel(x, W_enc, W_dec, b_enc, b_dec):
    W_enc_bf = W_enc.astype(jnp.bfloat16)
    W_dec_bf = W_dec.astype(jnp.bfloat16)
    recs, ss, cols = [], [], []
    for i in range(PIPE):
        xi = jax.lax.slice_in_dim(x, i * BCH, (i + 1) * BCH, axis=0)
        pre = _encode(xi, W_enc_bf, b_enc, b_dec, bm=ENC_BM, bn=ENC_BN)
        tau = jnp.broadcast_to(_sc_select(pre)[:, None], (BCH, 128))
        rec, s, col = _decode(pre, tau, W_dec_bf, xi, b_dec,
                              bm=DEC_BM, bk=DEC_BK)
        recs.append(rec)
        ss.append(s)
        cols.append(col)
    recons = jnp.concatenate(recs, axis=0)
    s = sum(ss)
    col = sum(cols)
    s0 = s[0, 0]
    s1 = s[1, 0]
    colsum = col[0]
    denom = jnp.float32(B * D_IN)
    mse = s0 / denom
    mse_naive = (s1 - jnp.sum(colsum * colsum) / B) / denom
    mse_loss = mse / mse_naive
    aux_loss = jnp.asarray(0.0, dtype=jnp.float32)
    loss = mse_loss + aux_loss
    return recons, loss, mse_loss, aux_loss
